# Initial kernel scaffold; baseline (speedup 1.0000x reference)
import jax
import jax.numpy as jnp
from jax.experimental import pallas as pl


def kernel(z, Wr, br, W1, b1, W2, b2):
    raise NotImplementedError("write your pallas kernel here")


# speedup vs baseline: 6.9596x; 6.9596x over previous
import functools
import math

import jax
import jax.numpy as jnp
from jax.experimental import pallas as pl
from jax.experimental.pallas import tpu as pltpu

T = 8192
D = 768
E = 8
H = 256
K = 2

TB = 512


def _moe_body(z_ref, wr_ref, br_ref, w1_ref, b1_ref, w2_ref, b2_ref, out_ref):
    zt = z_ref[...]
    zb = zt.astype(jnp.bfloat16)

    logits = (
        jnp.dot(zb, wr_ref[...].astype(jnp.bfloat16),
                preferred_element_type=jnp.float32)
        + br_ref[...][None, :]
    )
    e_ids = jax.lax.broadcasted_iota(jnp.int32, (TB, E), 1)
    e1 = jnp.argmax(logits, axis=1).astype(jnp.int32)
    mask1 = e_ids == e1[:, None]
    neg = jnp.where(mask1, -jnp.inf, logits)
    e2 = jnp.argmax(neg, axis=1).astype(jnp.int32)
    mask2 = e_ids == e2[:, None]
    m1 = jnp.max(logits, axis=1, keepdims=True)
    p = jnp.exp(logits - m1)
    w_raw = jnp.where(mask1 | mask2, p, 0.0)
    w = w_raw / jnp.sum(w_raw, axis=1, keepdims=True)

    h = jnp.dot(zb, w1_ref[...], preferred_element_type=jnp.float32)
    h = h + b1_ref[...][None, :]
    h = 0.5 * h * (1.0 + jax.lax.erf(h * (1.0 / math.sqrt(2.0))))
    hs = jnp.concatenate(
        [h[:, e * H:(e + 1) * H] * w[:, e:e + 1] for e in range(E)], axis=1
    ).astype(jnp.bfloat16)
    out = jnp.dot(hs, w2_ref[...], preferred_element_type=jnp.float32)
    out = out + jnp.dot(w, b2_ref[...], preferred_element_type=jnp.float32,
                        precision=jax.lax.Precision.HIGHEST)
    out_ref[...] = out


@jax.jit
def _moe(z, Wr, br, w1cat, b1flat, w2cat, b2):
    grid = (T // TB,)
    return pl.pallas_call(
        _moe_body,
        grid=grid,
        in_specs=[
            pl.BlockSpec((TB, D), lambda i: (i, 0)),
            pl.BlockSpec((D, E), lambda i: (0, 0)),
            pl.BlockSpec((E,), lambda i: (0,)),
            pl.BlockSpec((D, E * H), lambda i: (0, 0)),
            pl.BlockSpec((E * H,), lambda i: (0,)),
            pl.BlockSpec((E * H, D), lambda i: (0, 0)),
            pl.BlockSpec((E, D), lambda i: (0, 0)),
        ],
        out_specs=pl.BlockSpec((TB, D), lambda i: (i, 0)),
        out_shape=jax.ShapeDtypeStruct((T, D), jnp.float32),
        compiler_params=pltpu.CompilerParams(
            dimension_semantics=("arbitrary",),
        ),
    )(z, Wr, br, w1cat, b1flat, w2cat, b2)


def kernel(z, Wr, br, W1, b1, W2, b2):
    w1cat = jnp.transpose(W1, (1, 0, 2)).reshape(D, E * H).astype(jnp.bfloat16)
    w2cat = W2.reshape(E * H, D).astype(jnp.bfloat16)
    b1flat = b1.reshape(E * H)
    return _moe(z, Wr, br, w1cat, b1flat, w2cat, b2)


# speedup vs baseline: 7.2553x; 1.0425x over previous
import functools
import math

import jax
import jax.numpy as jnp
from jax.experimental import pallas as pl
from jax.experimental.pallas import tpu as pltpu

T = 8192
D = 768
E = 8
H = 256
K = 2

TB = 512


def _moe_body(z_ref, wr_ref, br_ref, w1_ref, b1_ref, w2_ref, b2_ref, out_ref):
    zt = z_ref[...]
    zb = zt.astype(jnp.bfloat16)

    logits = (
        jnp.dot(zb, wr_ref[...].astype(jnp.bfloat16),
                preferred_element_type=jnp.float32)
        + br_ref[...][None, :]
    )
    e_ids = jax.lax.broadcasted_iota(jnp.int32, (TB, E), 1)
    e1 = jnp.argmax(logits, axis=1).astype(jnp.int32)
    mask1 = e_ids == e1[:, None]
    neg = jnp.where(mask1, -jnp.inf, logits)
    e2 = jnp.argmax(neg, axis=1).astype(jnp.int32)
    mask2 = e_ids == e2[:, None]
    m1 = jnp.max(logits, axis=1, keepdims=True)
    p = jnp.exp(logits - m1)
    w_raw = jnp.where(mask1 | mask2, p, 0.0)
    w = w_raw / jnp.sum(w_raw, axis=1, keepdims=True)

    h = jnp.dot(zb, w1_ref[...], preferred_element_type=jnp.float32)
    h = h + b1_ref[...][None, :]
    h = 0.5 * h * (1.0 + jax.lax.erf(h * (1.0 / math.sqrt(2.0))))
    blk = (jax.lax.broadcasted_iota(jnp.int32, (E, E * H), 1) // H
           == jax.lax.broadcasted_iota(jnp.int32, (E, E * H), 0))
    g = jnp.dot(w.astype(jnp.bfloat16), blk.astype(jnp.bfloat16),
                preferred_element_type=jnp.float32)
    hs = (h * g).astype(jnp.bfloat16)
    out = jnp.dot(hs, w2_ref[...], preferred_element_type=jnp.float32)
    out = out + jnp.dot(w.astype(jnp.bfloat16),
                        b2_ref[...].astype(jnp.bfloat16),
                        preferred_element_type=jnp.float32)
    out_ref[...] = out


@jax.jit
def _moe(z, Wr, br, w1cat, b1flat, w2cat, b2):
    grid = (T // TB,)
    return pl.pallas_call(
        _moe_body,
        grid=grid,
        in_specs=[
            pl.BlockSpec((TB, D), lambda i: (i, 0)),
            pl.BlockSpec((D, E), lambda i: (0, 0)),
            pl.BlockSpec((E,), lambda i: (0,)),
            pl.BlockSpec((D, E * H), lambda i: (0, 0)),
            pl.BlockSpec((E * H,), lambda i: (0,)),
            pl.BlockSpec((E * H, D), lambda i: (0, 0)),
            pl.BlockSpec((E, D), lambda i: (0, 0)),
        ],
        out_specs=pl.BlockSpec((TB, D), lambda i: (i, 0)),
        out_shape=jax.ShapeDtypeStruct((T, D), jnp.float32),
        compiler_params=pltpu.CompilerParams(
            dimension_semantics=("arbitrary",),
        ),
    )(z, Wr, br, w1cat, b1flat, w2cat, b2)


def kernel(z, Wr, br, W1, b1, W2, b2):
    w1cat = jnp.transpose(W1, (1, 0, 2)).reshape(D, E * H).astype(jnp.bfloat16)
    w2cat = W2.reshape(E * H, D).astype(jnp.bfloat16)
    b1flat = b1.reshape(E * H)
    return _moe(z, Wr, br, w1cat, b1flat, w2cat, b2)


# speedup vs baseline: 7.4795x; 1.0309x over previous
import functools
import math

import jax
import jax.numpy as jnp
from jax.experimental import pallas as pl
from jax.experimental.pallas import tpu as pltpu

T = 8192
D = 768
E = 8
H = 256
K = 2

TB = 1024


def _moe_body(z_ref, wr_ref, br_ref, w1_ref, b1_ref, w2_ref, b2_ref, out_ref):
    zt = z_ref[...]
    zb = zt.astype(jnp.bfloat16)

    logits = (
        jnp.dot(zb, wr_ref[...].astype(jnp.bfloat16),
                preferred_element_type=jnp.float32)
        + br_ref[...][None, :]
    )
    e_ids = jax.lax.broadcasted_iota(jnp.int32, (TB, E), 1)
    e1 = jnp.argmax(logits, axis=1).astype(jnp.int32)
    mask1 = e_ids == e1[:, None]
    neg = jnp.where(mask1, -jnp.inf, logits)
    e2 = jnp.argmax(neg, axis=1).astype(jnp.int32)
    mask2 = e_ids == e2[:, None]
    m1 = jnp.max(logits, axis=1, keepdims=True)
    p = jnp.exp(logits - m1)
    w_raw = jnp.where(mask1 | mask2, p, 0.0)
    w = w_raw / jnp.sum(w_raw, axis=1, keepdims=True)

    h = jnp.dot(zb, w1_ref[...], preferred_element_type=jnp.float32)
    h = h + b1_ref[...][None, :]
    h = 0.5 * h * (1.0 + jax.lax.erf(h * (1.0 / math.sqrt(2.0))))
    blk = (jax.lax.broadcasted_iota(jnp.int32, (E, E * H), 1) // H
           == jax.lax.broadcasted_iota(jnp.int32, (E, E * H), 0))
    g = jnp.dot(w.astype(jnp.bfloat16), blk.astype(jnp.bfloat16),
                preferred_element_type=jnp.float32)
    hs = (h * g).astype(jnp.bfloat16)
    out = jnp.dot(hs, w2_ref[...], preferred_element_type=jnp.float32)
    out = out + jnp.dot(w.astype(jnp.bfloat16),
                        b2_ref[...].astype(jnp.bfloat16),
                        preferred_element_type=jnp.float32)
    out_ref[...] = out


@jax.jit
def _moe(z, Wr, br, w1cat, b1flat, w2cat, b2):
    grid = (T // TB,)
    return pl.pallas_call(
        _moe_body,
        grid=grid,
        in_specs=[
            pl.BlockSpec((TB, D), lambda i: (i, 0)),
            pl.BlockSpec((D, E), lambda i: (0, 0)),
            pl.BlockSpec((E,), lambda i: (0,)),
            pl.BlockSpec((D, E * H), lambda i: (0, 0)),
            pl.BlockSpec((E * H,), lambda i: (0,)),
            pl.BlockSpec((E * H, D), lambda i: (0, 0)),
            pl.BlockSpec((E, D), lambda i: (0, 0)),
        ],
        out_specs=pl.BlockSpec((TB, D), lambda i: (i, 0)),
        out_shape=jax.ShapeDtypeStruct((T, D), jnp.float32),
        compiler_params=pltpu.CompilerParams(
            dimension_semantics=("arbitrary",),
        ),
    )(z, Wr, br, w1cat, b1flat, w2cat, b2)


def kernel(z, Wr, br, W1, b1, W2, b2):
    w1cat = jnp.transpose(W1, (1, 0, 2)).reshape(D, E * H).astype(jnp.bfloat16)
    w2cat = W2.reshape(E * H, D).astype(jnp.bfloat16)
    b1flat = b1.reshape(E * H)
    return _moe(z, Wr, br, w1cat, b1flat, w2cat, b2)


# speedup vs baseline: 7.8793x; 1.0535x over previous
import functools
import math

import jax
import jax.numpy as jnp
from jax.experimental import pallas as pl
from jax.experimental.pallas import tpu as pltpu

T = 8192
D = 768
E = 8
H = 256
K = 2

TB = 1024


def _moe_body(z_ref, wr_ref, br_ref, w1_ref, b1_ref, w2_ref, b2_ref, out_ref,
              w1s, w2s):
    @pl.when(pl.program_id(0) == 0)
    def _prep():
        for e in range(E):
            w1s[:, e * H:(e + 1) * H] = w1_ref[e].astype(jnp.bfloat16)
        w2s[...] = w2_ref[...].astype(jnp.bfloat16)

    zt = z_ref[...]
    zb = zt.astype(jnp.bfloat16)

    logits = (
        jnp.dot(zb, wr_ref[...].astype(jnp.bfloat16),
                preferred_element_type=jnp.float32)
        + br_ref[...][None, :]
    )
    e_ids = jax.lax.broadcasted_iota(jnp.int32, (TB, E), 1)
    e1 = jnp.argmax(logits, axis=1).astype(jnp.int32)
    mask1 = e_ids == e1[:, None]
    neg = jnp.where(mask1, -jnp.inf, logits)
    e2 = jnp.argmax(neg, axis=1).astype(jnp.int32)
    mask2 = e_ids == e2[:, None]
    m1 = jnp.max(logits, axis=1, keepdims=True)
    p = jnp.exp(logits - m1)
    w_raw = jnp.where(mask1 | mask2, p, 0.0)
    w = w_raw / jnp.sum(w_raw, axis=1, keepdims=True)

    h = jnp.dot(zb, w1s[...], preferred_element_type=jnp.float32)
    h = h + b1_ref[...][None, :]
    h = 0.5 * h * (1.0 + jax.lax.erf(h * (1.0 / math.sqrt(2.0))))
    blk = (jax.lax.broadcasted_iota(jnp.int32, (E, E * H), 1) // H
           == jax.lax.broadcasted_iota(jnp.int32, (E, E * H), 0))
    g = jnp.dot(w.astype(jnp.bfloat16), blk.astype(jnp.bfloat16),
                preferred_element_type=jnp.float32)
    hs = (h * g).astype(jnp.bfloat16)
    out = jnp.dot(hs, w2s[...], preferred_element_type=jnp.float32)
    out = out + jnp.dot(w.astype(jnp.bfloat16),
                        b2_ref[...].astype(jnp.bfloat16),
                        preferred_element_type=jnp.float32)
    out_ref[...] = out


@jax.jit
def _moe(z, Wr, br, W1, b1flat, w2flat, b2):
    grid = (T // TB,)
    return pl.pallas_call(
        _moe_body,
        grid=grid,
        in_specs=[
            pl.BlockSpec((TB, D), lambda i: (i, 0)),
            pl.BlockSpec((D, E), lambda i: (0, 0)),
            pl.BlockSpec((E,), lambda i: (0,)),
            pl.BlockSpec((E, D, H), lambda i: (0, 0, 0)),
            pl.BlockSpec((E * H,), lambda i: (0,)),
            pl.BlockSpec((E * H, D), lambda i: (0, 0)),
            pl.BlockSpec((E, D), lambda i: (0, 0)),
        ],
        out_specs=pl.BlockSpec((TB, D), lambda i: (i, 0)),
        out_shape=jax.ShapeDtypeStruct((T, D), jnp.float32),
        scratch_shapes=[
            pltpu.VMEM((D, E * H), jnp.bfloat16),
            pltpu.VMEM((E * H, D), jnp.bfloat16),
        ],
        compiler_params=pltpu.CompilerParams(
            dimension_semantics=("arbitrary",),
        ),
    )(z, Wr, br, W1, b1flat, w2flat, b2)


def kernel(z, Wr, br, W1, b1, W2, b2):
    return _moe(z, Wr, br, W1, b1.reshape(E * H), W2.reshape(E * H, D), b2)


# speedup vs baseline: 7.8954x; 1.0020x over previous
import functools
import math

import jax
import jax.numpy as jnp
from jax.experimental import pallas as pl
from jax.experimental.pallas import tpu as pltpu

T = 8192
D = 768
E = 8
H = 256
K = 2

TB = 1024


def _moe_body(z_ref, wr_ref, br_ref, w1_ref, b1_ref, w2_ref, b2_ref, out_ref,
              w1s, w2s, gb_rhs):
    @pl.when(pl.program_id(0) == 0)
    def _prep():
        for e in range(E):
            w1s[:, e * H:(e + 1) * H] = w1_ref[e].astype(jnp.bfloat16)
        w2s[...] = w2_ref[...].astype(jnp.bfloat16)
        blk = (jax.lax.broadcasted_iota(jnp.int32, (E, E * H), 1) // H
               == jax.lax.broadcasted_iota(jnp.int32, (E, E * H), 0))
        gb_rhs[:, :E * H] = jnp.where(blk, 0.5, 0.0).astype(jnp.bfloat16)
        gb_rhs[:, E * H:] = b2_ref[...].astype(jnp.bfloat16)

    zt = z_ref[...]
    zb = zt.astype(jnp.bfloat16)

    logits = (
        jnp.dot(zb, wr_ref[...].astype(jnp.bfloat16),
                preferred_element_type=jnp.float32)
        + br_ref[...][None, :]
    )
    e_ids = jax.lax.broadcasted_iota(jnp.int32, (TB, E), 1)
    e1 = jnp.argmax(logits, axis=1).astype(jnp.int32)
    mask1 = e_ids == e1[:, None]
    neg = jnp.where(mask1, -jnp.inf, logits)
    e2 = jnp.argmax(neg, axis=1).astype(jnp.int32)
    mask2 = e_ids == e2[:, None]
    m1 = jnp.max(logits, axis=1, keepdims=True)
    p = jnp.exp(logits - m1)
    w_raw = jnp.where(mask1 | mask2, p, 0.0)
    w = w_raw / jnp.sum(w_raw, axis=1, keepdims=True)

    h = jnp.dot(zb, w1s[...], preferred_element_type=jnp.float32)
    h = h + b1_ref[...][None, :]
    u = 1.0 + jax.lax.erf(h * (1.0 / math.sqrt(2.0)))
    wb = w.astype(jnp.bfloat16)
    gb = jnp.dot(wb, gb_rhs[...], preferred_element_type=jnp.float32)
    g = gb[:, :E * H]
    hs = (h * u * g).astype(jnp.bfloat16)
    out = jnp.dot(hs, w2s[...], preferred_element_type=jnp.float32)
    out_ref[...] = out + gb[:, E * H:]


@jax.jit
def _moe(z, Wr, br, W1, b1flat, w2flat, b2):
    grid = (T // TB,)
    return pl.pallas_call(
        _moe_body,
        grid=grid,
        in_specs=[
            pl.BlockSpec((TB, D), lambda i: (i, 0)),
            pl.BlockSpec((D, E), lambda i: (0, 0)),
            pl.BlockSpec((E,), lambda i: (0,)),
            pl.BlockSpec((E, D, H), lambda i: (0, 0, 0)),
            pl.BlockSpec((E * H,), lambda i: (0,)),
            pl.BlockSpec((E * H, D), lambda i: (0, 0)),
            pl.BlockSpec((E, D), lambda i: (0, 0)),
        ],
        out_specs=pl.BlockSpec((TB, D), lambda i: (i, 0)),
        out_shape=jax.ShapeDtypeStruct((T, D), jnp.float32),
        scratch_shapes=[
            pltpu.VMEM((D, E * H), jnp.bfloat16),
            pltpu.VMEM((E * H, D), jnp.bfloat16),
            pltpu.VMEM((E, E * H + D), jnp.bfloat16),
        ],
        compiler_params=pltpu.CompilerParams(
            dimension_semantics=("arbitrary",),
        ),
    )(z, Wr, br, W1, b1flat, w2flat, b2)


def kernel(z, Wr, br, W1, b1, W2, b2):
    return _moe(z, Wr, br, W1, b1.reshape(E * H), W2.reshape(E * H, D), b2)
